# baseline (speedup 1.0000x reference)
import functools

import jax
import jax.numpy as jnp
from jax import lax
from jax.experimental import pallas as pl
from jax.experimental.pallas import tpu as pltpu
from jax.experimental.pallas import tpu_sc as plsc

_N = 10000
_D = 128
_E = 320000

_NC = 2
_NS = 16
_NW = _NC * _NS
_C = 128
_NPAD = 10240
_RPS = _NPAD // _NS
_NCH = 236
_NCH0 = 320
_NCH1 = 2 * _NCH - _NCH0
_EPAD = _NW * _NCH * _C
_BS = 400
_NB = _N // _BS


def _mm_body(nf_ref, a_ref, b_ref, out_ref):
    nf = nf_ref[...]
    for h in range(3):
        out_ref[h] = jnp.dot(nf, a_ref[h], preferred_element_type=jnp.float32)
    t = jnp.dot(nf, a_ref[3], preferred_element_type=jnp.float32)
    out_ref[3] = jnp.dot(t, b_ref[...], preferred_element_type=jnp.float32)


def _fin_body(p_ref, d_ref, o_ref):
    o_ref[...] = jnp.maximum(p_ref[0] + p_ref[1] + d_ref[...], 0.0)


_ROWS_PER_HOP = _E // _C
_NROWS = 3 * _ROWS_PER_HOP


def _sc_spmm(x_flat, ei0, ei1, ei2, v0, v1, v2):
    mesh = plsc.VectorSubcoreMesh(core_axis_name="c", subcore_axis_name="s",
                                  num_cores=_NC, num_subcores=_NS)

    @functools.partial(
        pl.kernel,
        out_type=jax.ShapeDtypeStruct((_NC, _NPAD, _D), jnp.float32),
        mesh=mesh,
        scratch_types=[
            pltpu.VMEM_SHARED((_NPAD, _D), jnp.float32),
            pltpu.VMEM((4, _C), jnp.int32),
            pltpu.VMEM((4, _C), jnp.int32),
            pltpu.VMEM((4, _C), jnp.float32),
            pltpu.VMEM((2, _C, _D), jnp.float32),
            pltpu.SemaphoreType.DMA((4,)),
            pltpu.SemaphoreType.DMA((2,)),
            pltpu.SemaphoreType.DMA((2,)),
        ],
    )
    def body(x_hbm, e0_hbm, e1_hbm, e2_hbm, v0_hbm, v1_hbm, v2_hbm, out_hbm,
             acc, src_v, dst_v, val_v, rows_v, sem_i, sem_g, sem_s):
        c = lax.axis_index("c")
        s = lax.axis_index("s")
        count = jnp.where(c == 0, _NCH0, _NCH1)
        base = jnp.where(c == 0, s * _NCH0, _NS * _NCH0 + s * _NCH1)

        @pl.loop(0, _C)
        def _zero(e):
            for l in range(8):
                rows_v[0, e, pl.ds(l * 16, 16)] = jnp.zeros((16,), jnp.float32)

        for j in range(_RPS // _C):
            pltpu.sync_copy(rows_v.at[0], acc.at[pl.ds(s * _RPS + j * _C, _C)])
        plsc.subcore_barrier()

        def idx_start(chunk, ib):
            row = base + chunk
            pad = row >= _NROWS

            def copy_hop(e_hbm, v_hbm, rr):
                off = rr * _C
                pltpu.async_copy(e_hbm.at[1, pl.ds(off, _C)], src_v.at[ib],
                                 sem_i.at[ib])
                pltpu.async_copy(e_hbm.at[0, pl.ds(off, _C)], dst_v.at[ib],
                                 sem_i.at[ib])
                pltpu.async_copy(v_hbm.at[pl.ds(off, _C)], val_v.at[ib],
                                 sem_i.at[ib])

            @pl.when((row < _ROWS_PER_HOP) | pad)
            def _():
                copy_hop(e0_hbm, v0_hbm, jnp.where(pad, 0, row))

            @pl.when((row >= _ROWS_PER_HOP) & (row < 2 * _ROWS_PER_HOP))
            def _():
                copy_hop(e1_hbm, v1_hbm, row - _ROWS_PER_HOP)

            @pl.when((row >= 2 * _ROWS_PER_HOP) & (row < _NROWS))
            def _():
                copy_hop(e2_hbm, v2_hbm, row - 2 * _ROWS_PER_HOP)

        def idx_wait(chunk, ib):
            pltpu.make_async_copy(e0_hbm.at[1, pl.ds(0, _C)], src_v.at[ib],
                                  sem_i.at[ib]).wait()
            pltpu.make_async_copy(e0_hbm.at[0, pl.ds(0, _C)], dst_v.at[ib],
                                  sem_i.at[ib]).wait()
            pltpu.make_async_copy(v0_hbm.at[pl.ds(0, _C)], val_v.at[ib],
                                  sem_i.at[ib]).wait()

        def fix_chunk(chunk, ib):
            row = base + chunk
            off = jnp.where(
                row >= _NROWS, 0,
                jnp.where(row >= 2 * _ROWS_PER_HOP, 2 * _N,
                          jnp.where(row >= _ROWS_PER_HOP, _N, 0))).astype(
                              jnp.int32)

            @pl.when(off != 0)
            def _():
                for l in range(8):
                    sl = pl.ds(l * 16, 16)
                    src_v[ib, sl] = src_v[ib, sl] + off

            @pl.when(row >= _NROWS)
            def _():
                for l in range(8):
                    val_v[ib, pl.ds(l * 16, 16)] = jnp.zeros((16,),
                                                             jnp.float32)

        def gather_start(ib, rb):
            pltpu.async_copy(x_hbm.at[src_v.at[ib]], rows_v.at[rb], sem_g.at[rb])

        def gather_wait(ib, rb):
            pltpu.make_async_copy(x_hbm.at[src_v.at[ib]], rows_v.at[rb],
                                  sem_g.at[rb]).wait()

        def scat_start(ib, rb):
            pltpu.async_copy(rows_v.at[rb], acc.at[dst_v.at[ib]], sem_s.at[rb],
                             add=True)

        def scat_wait(ib, rb):
            pltpu.make_async_copy(rows_v.at[rb], acc.at[dst_v.at[ib]],
                                  sem_s.at[rb]).wait()

        def scale(ib, rb):
            @pl.loop(0, _C // 16)
            def _sc(k):
                v16 = val_v[ib, pl.ds(k * 16, 16)]
                for j in range(16):
                    e = k * 16 + j
                    v = v16[j]
                    for l in range(8):
                        sl = pl.ds(l * 16, 16)
                        rows_v[rb, e, sl] = rows_v[rb, e, sl] * v

        for j in range(3):
            idx_start(j, j)
        idx_wait(0, 0)
        fix_chunk(0, 0)
        gather_start(0, 0)

        @pl.loop(0, count, step=4)
        def _main(g):
            for j in range(4):
                chunk = g + j
                rb = j % 2
                nrb = (j + 1) % 2
                ib = j
                nib = (j + 1) % 4
                pib = (j + 3) % 4

                @pl.when(chunk >= 1)
                def _():
                    scat_wait(pib, nrb)

                @pl.when(chunk + 1 < count)
                def _():
                    idx_wait(chunk + 1, nib)
                    fix_chunk(chunk + 1, nib)
                    gather_start(nib, nrb)

                gather_wait(ib, rb)
                scale(ib, rb)
                scat_start(ib, rb)

                @pl.when(chunk + 3 < count)
                def _():
                    idx_start(chunk + 3, pib)

        scat_wait(3, 1)
        plsc.subcore_barrier()
        pltpu.sync_copy(acc.at[pl.ds(s * _RPS, _RPS)],
                        out_hbm.at[c, pl.ds(s * _RPS, _RPS)])

    return body(x_flat, ei0, ei1, ei2, v0, v1, v2)


def kernel(node_features, edge_index0, edge_index1, edge_index2,
           adj_val0, adj_val1, adj_val2,
           W_embed, W_feat0, W_feat1, W_feat2,
           W_emb0, W_emb1, W_emb2, alpha):
    a_stack = jnp.stack([W_feat0, W_feat1, W_feat2, W_embed])
    b_comb = jnp.float32(alpha) * (W_emb0 + W_emb1 + W_emb2)

    x = pl.pallas_call(
        _mm_body,
        grid=(_NB,),
        in_specs=[
            pl.BlockSpec((_BS, _D), lambda i: (i, 0)),
            pl.BlockSpec((4, _D, _D), lambda i: (0, 0, 0)),
            pl.BlockSpec((_D, _D), lambda i: (0, 0)),
        ],
        out_specs=pl.BlockSpec((4, _BS, _D), lambda i: (0, i, 0)),
        out_shape=jax.ShapeDtypeStruct((4, _N, _D), jnp.float32),
    )(node_features, a_stack, b_comb)
    x_flat = x.reshape(4 * _N, _D)

    partials = _sc_spmm(x_flat, edge_index0, edge_index1, edge_index2,
                        adj_val0, adj_val1, adj_val2)

    return pl.pallas_call(
        _fin_body,
        grid=(_NB,),
        in_specs=[
            pl.BlockSpec((2, _BS, _D), lambda i: (0, i, 0)),
            pl.BlockSpec((_BS, _D), lambda i: (3 * _NB + i, 0)),
        ],
        out_specs=pl.BlockSpec((_BS, _D), lambda i: (i, 0)),
        out_shape=jax.ShapeDtypeStruct((_N, _D), jnp.float32),
    )(partials, x_flat)


# speedup vs baseline: 1.2915x; 1.2915x over previous
import functools

import jax
import jax.numpy as jnp
from jax import lax
from jax.experimental import pallas as pl
from jax.experimental.pallas import tpu as pltpu
from jax.experimental.pallas import tpu_sc as plsc

_N = 10000
_D = 128
_E = 320000

_NC = 2
_NS = 16
_NW = _NC * _NS
_C = 128
_NPAD = 10240
_RPS = _NPAD // _NS
_NCH = 236
_NCH0 = 236
_NCH1 = 2 * _NCH - _NCH0
_EPAD = _NW * _NCH * _C
_BS = 400
_NB = _N // _BS


def _mm_body(nf_ref, a_ref, b_ref, out_ref):
    nf = nf_ref[...]
    for h in range(3):
        out_ref[h] = jnp.dot(nf, a_ref[h], preferred_element_type=jnp.float32)
    t = jnp.dot(nf, a_ref[3], preferred_element_type=jnp.float32)
    out_ref[3] = jnp.dot(t, b_ref[...], preferred_element_type=jnp.float32)


def _fin_body(p_ref, d_ref, o_ref):
    o_ref[...] = jnp.maximum(p_ref[0] + p_ref[1] + d_ref[...], 0.0)


_ROWS_PER_HOP = _E // _C
_NROWS = 3 * _ROWS_PER_HOP


def _sc_spmm(x_flat, ei0, ei1, ei2, v0, v1, v2):
    mesh = plsc.VectorSubcoreMesh(core_axis_name="c", subcore_axis_name="s",
                                  num_cores=_NC, num_subcores=_NS)

    @functools.partial(
        pl.kernel,
        out_type=jax.ShapeDtypeStruct((_NC, _NPAD, _D), jnp.float32),
        mesh=mesh,
        scratch_types=[
            pltpu.VMEM_SHARED((_NPAD, _D), jnp.float32),
            pltpu.VMEM((4, _C), jnp.int32),
            pltpu.VMEM((4, _C), jnp.int32),
            pltpu.VMEM((4, _C), jnp.float32),
            pltpu.VMEM((2, _C, _D), jnp.float32),
            pltpu.SemaphoreType.DMA((4,)),
            pltpu.SemaphoreType.DMA((2,)),
            pltpu.SemaphoreType.DMA((2,)),
        ],
    )
    def body(x_hbm, e0_hbm, e1_hbm, e2_hbm, v0_hbm, v1_hbm, v2_hbm, out_hbm,
             acc, src_v, dst_v, val_v, rows_v, sem_i, sem_g, sem_s):
        c = lax.axis_index("c")
        s = lax.axis_index("s")
        count = jnp.where(c == 0, _NCH0, _NCH1)
        base = jnp.where(c == 0, s * _NCH0, _NS * _NCH0 + s * _NCH1)

        @pl.loop(0, _C)
        def _zero(e):
            for l in range(8):
                rows_v[0, e, pl.ds(l * 16, 16)] = jnp.zeros((16,), jnp.float32)

        for j in range(_RPS // _C):
            pltpu.sync_copy(rows_v.at[0], acc.at[pl.ds(s * _RPS + j * _C, _C)])
        plsc.subcore_barrier()

        def idx_start(chunk, ib):
            row = base + chunk
            pad = row >= _NROWS

            def copy_hop(e_hbm, v_hbm, rr):
                off = rr * _C
                pltpu.async_copy(e_hbm.at[1, pl.ds(off, _C)], src_v.at[ib],
                                 sem_i.at[ib])
                pltpu.async_copy(e_hbm.at[0, pl.ds(off, _C)], dst_v.at[ib],
                                 sem_i.at[ib])
                pltpu.async_copy(v_hbm.at[pl.ds(off, _C)], val_v.at[ib],
                                 sem_i.at[ib])

            @pl.when((row < _ROWS_PER_HOP) | pad)
            def _():
                copy_hop(e0_hbm, v0_hbm, jnp.where(pad, 0, row))

            @pl.when((row >= _ROWS_PER_HOP) & (row < 2 * _ROWS_PER_HOP))
            def _():
                copy_hop(e1_hbm, v1_hbm, row - _ROWS_PER_HOP)

            @pl.when((row >= 2 * _ROWS_PER_HOP) & (row < _NROWS))
            def _():
                copy_hop(e2_hbm, v2_hbm, row - 2 * _ROWS_PER_HOP)

        def idx_wait(chunk, ib):
            pltpu.make_async_copy(e0_hbm.at[1, pl.ds(0, _C)], src_v.at[ib],
                                  sem_i.at[ib]).wait()
            pltpu.make_async_copy(e0_hbm.at[0, pl.ds(0, _C)], dst_v.at[ib],
                                  sem_i.at[ib]).wait()
            pltpu.make_async_copy(v0_hbm.at[pl.ds(0, _C)], val_v.at[ib],
                                  sem_i.at[ib]).wait()

        def fix_chunk(chunk, ib):
            row = base + chunk
            off = jnp.where(
                row >= _NROWS, 0,
                jnp.where(row >= 2 * _ROWS_PER_HOP, 2 * _N,
                          jnp.where(row >= _ROWS_PER_HOP, _N, 0))).astype(
                              jnp.int32)

            @pl.when(off != 0)
            def _():
                for l in range(8):
                    sl = pl.ds(l * 16, 16)
                    src_v[ib, sl] = src_v[ib, sl] + off

            @pl.when(row >= _NROWS)
            def _():
                for l in range(8):
                    val_v[ib, pl.ds(l * 16, 16)] = jnp.zeros((16,),
                                                             jnp.float32)

        def gather_start(ib, rb):
            pltpu.async_copy(x_hbm.at[src_v.at[ib]], rows_v.at[rb], sem_g.at[rb])

        def gather_wait(ib, rb):
            pltpu.make_async_copy(x_hbm.at[src_v.at[ib]], rows_v.at[rb],
                                  sem_g.at[rb]).wait()

        def scat_start(ib, rb):
            pltpu.async_copy(rows_v.at[rb], acc.at[dst_v.at[ib]], sem_s.at[rb],
                             add=True)

        def scat_wait(ib, rb):
            pltpu.make_async_copy(rows_v.at[rb], acc.at[dst_v.at[ib]],
                                  sem_s.at[rb]).wait()

        def scale(ib, rb):
            @pl.loop(0, _C // 16)
            def _sc(k):
                v16 = val_v[ib, pl.ds(k * 16, 16)]
                for j in range(16):
                    e = k * 16 + j
                    v = v16[j]
                    for l in range(8):
                        sl = pl.ds(l * 16, 16)
                        rows_v[rb, e, sl] = rows_v[rb, e, sl] * v

        for j in range(3):
            idx_start(j, j)
        idx_wait(0, 0)
        fix_chunk(0, 0)
        gather_start(0, 0)

        @pl.loop(0, count, step=4)
        def _main(g):
            for j in range(4):
                chunk = g + j
                rb = j % 2
                nrb = (j + 1) % 2
                ib = j
                nib = (j + 1) % 4
                pib = (j + 3) % 4

                @pl.when(chunk >= 1)
                def _():
                    scat_wait(pib, nrb)

                @pl.when(chunk + 1 < count)
                def _():
                    idx_wait(chunk + 1, nib)
                    fix_chunk(chunk + 1, nib)
                    gather_start(nib, nrb)

                gather_wait(ib, rb)
                scale(ib, rb)
                scat_start(ib, rb)

                @pl.when(chunk + 3 < count)
                def _():
                    idx_start(chunk + 3, pib)

        scat_wait(3, 1)
        plsc.subcore_barrier()
        pltpu.sync_copy(acc.at[pl.ds(s * _RPS, _RPS)],
                        out_hbm.at[c, pl.ds(s * _RPS, _RPS)])

    return body(x_flat, ei0, ei1, ei2, v0, v1, v2)


def kernel(node_features, edge_index0, edge_index1, edge_index2,
           adj_val0, adj_val1, adj_val2,
           W_embed, W_feat0, W_feat1, W_feat2,
           W_emb0, W_emb1, W_emb2, alpha):
    a_stack = jnp.stack([W_feat0, W_feat1, W_feat2, W_embed])
    b_comb = jnp.float32(alpha) * (W_emb0 + W_emb1 + W_emb2)

    x = pl.pallas_call(
        _mm_body,
        grid=(_NB,),
        in_specs=[
            pl.BlockSpec((_BS, _D), lambda i: (i, 0)),
            pl.BlockSpec((4, _D, _D), lambda i: (0, 0, 0)),
            pl.BlockSpec((_D, _D), lambda i: (0, 0)),
        ],
        out_specs=pl.BlockSpec((4, _BS, _D), lambda i: (0, i, 0)),
        out_shape=jax.ShapeDtypeStruct((4, _N, _D), jnp.float32),
    )(node_features, a_stack, b_comb)
    x_flat = x.reshape(4 * _N, _D)

    partials = _sc_spmm(x_flat, edge_index0, edge_index1, edge_index2,
                        adj_val0, adj_val1, adj_val2)

    return pl.pallas_call(
        _fin_body,
        grid=(_NB,),
        in_specs=[
            pl.BlockSpec((2, _BS, _D), lambda i: (0, i, 0)),
            pl.BlockSpec((_BS, _D), lambda i: (3 * _NB + i, 0)),
        ],
        out_specs=pl.BlockSpec((_BS, _D), lambda i: (i, 0)),
        out_shape=jax.ShapeDtypeStruct((_N, _D), jnp.float32),
    )(partials, x_flat)
